# Initial kernel scaffold; baseline (speedup 1.0000x reference)
import jax
import jax.numpy as jnp
from jax.experimental import pallas as pl


def kernel(x, pos, edge_index):
    raise NotImplementedError("write your pallas kernel here")


# speedup vs baseline: 1.4665x; 1.4665x over previous
import functools

import jax
import jax.numpy as jnp
from jax import lax
from jax.experimental import pallas as pl
from jax.experimental.pallas import tpu as pltpu
from jax.experimental.pallas import tpu_sc as plsc

N = 10000
E = 320000
DF = 128
D = 144
DV = D // 16
NC = 2
NS = 16
NW = NC * NS
NPW = 320
C = 8000
G = 128
NFB = 3
ACC_ROWS = NFB * G
DUMMY = ACC_ROWS - 1
T_ROWS = 10304
OUT_ROWS = NW * NPW


def _gather16(v, idx):
    return lax.gather(
        v, idx[:, None],
        dimension_numbers=lax.GatherDimensionNumbers(
            offset_dims=(), collapsed_slice_dims=(0,), start_index_map=(0,)),
        slice_sizes=(1,), mode=lax.GatherScatterMode.PROMISE_IN_BOUNDS)


def _prefix_sum16(ones):
    iota = lax.iota(jnp.int32, 16)
    v = ones
    for step in (1, 2, 4, 8):
        g = _gather16(v, jnp.maximum(iota - step, 0))
        v = v + jnp.where(iota >= step, g, 0)
    return v


def _body(t_hbm, src_hbm, dst_hbm, out_hbm, ebuf_s, ebuf_d, cls, cld, rows,
          acc, sem):
    wid = lax.axis_index("s") * NC + lax.axis_index("c")
    base = wid * NPW
    neg_inf = jnp.full((16,), -jnp.inf, jnp.float32)

    def init_row(r, carry):
        for v in range(DV):
            acc[r, pl.ds(v * 16, 16)] = neg_inf
        return carry

    lax.fori_loop(0, ACC_ROWS, init_row, 0)

    def chunk_body(ck, carry):
        pltpu.sync_copy(src_hbm.at[pl.ds(ck * C, C)], ebuf_s)
        pltpu.sync_copy(dst_hbm.at[pl.ds(ck * C, C)], ebuf_d)

        def scan_body(i, cnt):
            off = i * 16
            dv = ebuf_d[pl.ds(off, 16)]
            sv = ebuf_s[pl.ds(off, 16)]
            ldv = dv - base
            m = (ldv >= 0) & (ldv < NPW)
            pfx = _prefix_sum16(jnp.where(m, 1, 0).astype(jnp.int32))
            pos = pfx + (cnt - 1)
            plsc.store_scatter(cls, [pos], sv, mask=m)
            plsc.store_scatter(cld, [pos], ldv, mask=m)
            return cnt + pfx[15]

        cnt = lax.fori_loop(0, C // 16, scan_body, jnp.int32(0))

        for j in range(G // 16):
            cls[pl.ds(cnt + j * 16, 16)] = jnp.zeros((16,), jnp.int32)
            cld[pl.ds(cnt + j * 16, 16)] = jnp.full((16,), DUMMY, jnp.int32)

        ngroups = (cnt + (G - 1)) // G

        def group_body(g, carry2):
            pltpu.async_copy(t_hbm.at[cls.at[pl.ds(g * G, G)]], rows,
                             sem).wait()

            def edge_body(eb, carry3):
                dvec = cld[pl.ds(g * G + eb * 16, 16)]
                for k in range(16):
                    d = dvec[k]
                    e = eb * 16 + k
                    for v in range(DV):
                        sl = pl.ds(v * 16, 16)
                        acc[d, sl] = jnp.maximum(acc[d, sl], rows[e, sl])
                return carry3

            lax.fori_loop(0, G // 16, edge_body, 0)
            return carry2

        lax.fori_loop(0, ngroups, group_body, 0)
        return carry

    lax.fori_loop(0, E // C, chunk_body, 0)

    for fb in range(NFB):
        pltpu.sync_copy(t_hbm.at[pl.ds(base + fb * G, G)], rows)

        def fin_body(e, carry):
            r = fb * G + e
            for v in range(DV):
                sl = pl.ds(v * 16, 16)
                a = acc[r, sl]
                t = rows[e, sl]
                if v < DF // 16:
                    acc[r, sl] = jnp.maximum(a, t)
                else:
                    acc[r, sl] = jnp.maximum(a - t, 0.0)
            return carry

        lax.fori_loop(0, G, fin_body, 0)

    pltpu.sync_copy(acc.at[pl.ds(0, NPW)], out_hbm.at[pl.ds(base, NPW)])


_mesh = plsc.VectorSubcoreMesh(core_axis_name="c", subcore_axis_name="s")

_sc_call = functools.partial(
    pl.kernel,
    mesh=_mesh,
    out_type=jax.ShapeDtypeStruct((OUT_ROWS, D), jnp.float32),
    scratch_types=[
        pltpu.VMEM((C,), jnp.int32),
        pltpu.VMEM((C,), jnp.int32),
        pltpu.VMEM((C + G,), jnp.int32),
        pltpu.VMEM((C + G,), jnp.int32),
        pltpu.VMEM((G, D), jnp.float32),
        pltpu.VMEM((ACC_ROWS, D), jnp.float32),
        pltpu.SemaphoreType.DMA,
    ],
    compiler_params=pltpu.CompilerParams(needs_layout_passes=False,
                                         use_tc_tiling_on_sc=False),
)(_body)


def kernel(x, pos, edge_index):
    src = edge_index[0].astype(jnp.int32)
    dst = edge_index[1].astype(jnp.int32)
    t = jnp.zeros((T_ROWS, D), jnp.float32)
    t = t.at[:N, :DF].set(x)
    t = t.at[:N, DF:DF + 3].set(pos)
    full = _sc_call(t, src, dst)
    return full[:N, :DF + 3]
